# baseline (speedup 1.0000x reference)
import functools

import jax
import jax.numpy as jnp
from jax import lax
from jax.experimental import pallas as pl
from jax.experimental.pallas import tpu as pltpu
from jax.experimental.pallas import tpu_sc as plsc

B = 16384
DIMS = (10, 10, 8, 8, 6, 6, 6, 6, 2, 2)
ROWS = (18, 19, 10, 11, 14, 6, 3, 9, 2, 2)
NF = len(DIMS)
D_OUT = sum(DIMS)

COL_OFF = []
_acc = 0
for _d in DIMS:
    COL_OFF.append(_acc)
    _acc += _d

TBL_BASE = []
_acc = 0
for _n, _d in zip(ROWS, DIMS):
    TBL_BASE.append(_acc)
    _acc += _n * _d
TBL_WORDS = _acc
TBL_PAD = (TBL_WORDS + 7) // 8 * 8

FIELD_OF_COL = []
for _f, _d in enumerate(DIMS):
    FIELD_OF_COL.extend([_f] * _d)

NC, NS, L = 2, 16, 16
NW = NC * NS
CHUNK = B // NW
NGRP = D_OUT // L
UNROLL = 4

_FMAP, _DVEC, _BJVEC = [], [], []
for _g in range(NGRP):
    cols = range(_g * L, (_g + 1) * L)
    fs = [FIELD_OF_COL[c] for c in cols]
    _FMAP.append(tuple(fs))
    _DVEC.append(tuple(DIMS[f] for f in fs))
    _BJVEC.append(tuple(TBL_BASE[f] + (c - COL_OFF[f])
                        for c, f in zip(cols, fs)))

_CONSTS = tuple(_FMAP) + tuple(_DVEC) + tuple(_BJVEC)
NCONST = len(_CONSTS)

_mesh = plsc.VectorSubcoreMesh(core_axis_name="c", subcore_axis_name="s")


@functools.partial(
    pl.kernel,
    out_type=jax.ShapeDtypeStruct((B, D_OUT), jnp.float32),
    mesh=_mesh,
    compiler_params=pltpu.CompilerParams(
        needs_layout_passes=False, use_tc_tiling_on_sc=False),
    scratch_types=[
        pltpu.VMEM((CHUNK, NF), jnp.int32),
        pltpu.VMEM((TBL_PAD,), jnp.float32),
        pltpu.VMEM((CHUNK, D_OUT), jnp.float32),
        pltpu.VMEM((NCONST, L), jnp.int32),
    ],
)
def _embed_sc(idx_hbm, tbl_hbm, cmat_hbm, out_hbm, stage_v, tbl_v, out_v,
              cmat_v):
    wid = lax.axis_index("s") * NC + lax.axis_index("c")
    base = wid * CHUNK

    pltpu.sync_copy(tbl_hbm, tbl_v)
    pltpu.sync_copy(cmat_hbm, cmat_v)
    pltpu.sync_copy(idx_hbm.at[pl.ds(base, CHUNK)], stage_v)

    fmap = [cmat_v[g] for g in range(NGRP)]
    dvec = [cmat_v[NGRP + g] for g in range(NGRP)]
    bjvec = [cmat_v[2 * NGRP + g] for g in range(NGRP)]

    def body(it, carry):
        row0 = it * UNROLL
        for r in range(UNROLL):
            b = row0 + r
            bsplat = jnp.full((L,), b, jnp.int32)
            for g in range(NGRP):
                raw = plsc.load_gather(stage_v, [bsplat, fmap[g]])
                tidx = raw * dvec[g] + bjvec[g]
                vals = plsc.load_gather(tbl_v, [tidx])
                out_v[b, pl.ds(g * L, L)] = vals
        return carry

    lax.fori_loop(0, CHUNK // UNROLL, body, 0)
    pltpu.sync_copy(out_v, out_hbm.at[pl.ds(base, CHUNK)])


def kernel(type1, type2, primary_color, secondary_color, shape, size,
           evolution_stage, habitat, legendary, mythical,
           type1_table, type2_table, primary_color_table,
           secondary_color_table, shape_table, size_table,
           evolution_stage_table, habitat_table, legendary_table,
           mythical_table):
    idx2 = jnp.stack(
        [x.astype(jnp.int32) for x in
         (type1, type2, primary_color, secondary_color, shape, size,
          evolution_stage, habitat, legendary, mythical)], axis=1)
    tables = (type1_table, type2_table, primary_color_table,
              secondary_color_table, shape_table, size_table,
              evolution_stage_table, habitat_table, legendary_table,
              mythical_table)
    tbl_flat = jnp.concatenate([t.reshape(-1) for t in tables])
    tbl_flat = jnp.pad(tbl_flat, (0, TBL_PAD - TBL_WORDS))
    cmat = jnp.asarray(_CONSTS, dtype=jnp.int32)
    return _embed_sc(idx2, tbl_flat, cmat)


# speedup vs baseline: 1.3199x; 1.3199x over previous
import functools

import jax
import jax.numpy as jnp
from jax import lax
from jax.experimental import pallas as pl
from jax.experimental.pallas import tpu as pltpu
from jax.experimental.pallas import tpu_sc as plsc

B = 16384
DIMS = (10, 10, 8, 8, 6, 6, 6, 6, 2, 2)
ROWS = (18, 19, 10, 11, 14, 6, 3, 9, 2, 2)
NF = len(DIMS)
D_OUT = sum(DIMS)

COL_OFF = []
_acc = 0
for _d in DIMS:
    COL_OFF.append(_acc)
    _acc += _d

TBL_BASE = []
_acc = 0
for _n, _d in zip(ROWS, DIMS):
    TBL_BASE.append(_acc)
    _acc += _n * _d
TBL_WORDS = _acc
TBL_PAD = (TBL_WORDS + 7) // 8 * 8

FIELD_OF_COL = []
for _f, _d in enumerate(DIMS):
    FIELD_OF_COL.extend([_f] * _d)

NC, NS, L = 2, 16, 16
NW = NC * NS
CHUNK = B // NW
NGRP = D_OUT // L
UNROLL = 4

_FMAP, _DVEC, _BJVEC = [], [], []
for _g in range(NGRP):
    cols = range(_g * L, (_g + 1) * L)
    fs = [FIELD_OF_COL[c] for c in cols]
    _FMAP.append(tuple(fs))
    _DVEC.append(tuple(DIMS[f] for f in fs))
    _BJVEC.append(tuple(TBL_BASE[f] + (c - COL_OFF[f])
                        for c, f in zip(cols, fs)))

_CONSTS = tuple(_FMAP) + tuple(_DVEC) + tuple(_BJVEC)
NCONST = len(_CONSTS)

_mesh = plsc.VectorSubcoreMesh(core_axis_name="c", subcore_axis_name="s")


@functools.partial(
    pl.kernel,
    out_type=jax.ShapeDtypeStruct((B, D_OUT), jnp.float32),
    mesh=_mesh,
    compiler_params=pltpu.CompilerParams(
        needs_layout_passes=False, use_tc_tiling_on_sc=False),
    scratch_types=[
        pltpu.VMEM((CHUNK, NF), jnp.int32),
        pltpu.VMEM((TBL_PAD,), jnp.float32),
        pltpu.VMEM((CHUNK, D_OUT), jnp.float32),
        pltpu.VMEM((NCONST, L), jnp.int32),
    ],
)
def _embed_sc(idx_hbm, tbl_hbm, cmat_hbm, out_hbm, stage_v, tbl_v, out_v,
              cmat_v):
    wid = lax.axis_index("s") * NC + lax.axis_index("c")
    base = wid * CHUNK

    pltpu.sync_copy(tbl_hbm, tbl_v)
    pltpu.sync_copy(cmat_hbm, cmat_v)
    pltpu.sync_copy(idx_hbm.at[pl.ds(base, CHUNK)], stage_v)

    fmap = [cmat_v[g] for g in range(NGRP)]
    dvec = [cmat_v[NGRP + g] for g in range(NGRP)]
    bjvec = [cmat_v[2 * NGRP + g] for g in range(NGRP)]

    @plsc.parallel_loop(0, CHUNK, unroll=UNROLL)
    def _row_loop(b):
        bsplat = jnp.full((L,), b, jnp.int32)
        for g in range(NGRP):
            raw = plsc.load_gather(stage_v, [bsplat, fmap[g]])
            tidx = raw * dvec[g] + bjvec[g]
            vals = plsc.load_gather(tbl_v, [tidx])
            out_v[b, pl.ds(g * L, L)] = vals
    pltpu.sync_copy(out_v, out_hbm.at[pl.ds(base, CHUNK)])


def kernel(type1, type2, primary_color, secondary_color, shape, size,
           evolution_stage, habitat, legendary, mythical,
           type1_table, type2_table, primary_color_table,
           secondary_color_table, shape_table, size_table,
           evolution_stage_table, habitat_table, legendary_table,
           mythical_table):
    idx2 = jnp.stack(
        [x.astype(jnp.int32) for x in
         (type1, type2, primary_color, secondary_color, shape, size,
          evolution_stage, habitat, legendary, mythical)], axis=1)
    tables = (type1_table, type2_table, primary_color_table,
              secondary_color_table, shape_table, size_table,
              evolution_stage_table, habitat_table, legendary_table,
              mythical_table)
    tbl_flat = jnp.concatenate([t.reshape(-1) for t in tables])
    tbl_flat = jnp.pad(tbl_flat, (0, TBL_PAD - TBL_WORDS))
    cmat = jnp.asarray(_CONSTS, dtype=jnp.int32)
    return _embed_sc(idx2, tbl_flat, cmat)


# speedup vs baseline: 1.3205x; 1.0004x over previous
import functools

import jax
import jax.numpy as jnp
from jax import lax
from jax.experimental import pallas as pl
from jax.experimental.pallas import tpu as pltpu
from jax.experimental.pallas import tpu_sc as plsc

B = 16384
DIMS = (10, 10, 8, 8, 6, 6, 6, 6, 2, 2)
ROWS = (18, 19, 10, 11, 14, 6, 3, 9, 2, 2)
NF = len(DIMS)
D_OUT = sum(DIMS)

COL_OFF = []
_acc = 0
for _d in DIMS:
    COL_OFF.append(_acc)
    _acc += _d

TBL_BASE = []
_acc = 0
for _n, _d in zip(ROWS, DIMS):
    TBL_BASE.append(_acc)
    _acc += _n * _d
TBL_WORDS = _acc
TBL_PAD = (TBL_WORDS + 7) // 8 * 8

FIELD_OF_COL = []
for _f, _d in enumerate(DIMS):
    FIELD_OF_COL.extend([_f] * _d)

NC, NS, L = 2, 16, 16
NW = NC * NS
CHUNK = B // NW
NGRP = D_OUT // L
UNROLL = 4

_FMAP, _DVEC, _BJVEC = [], [], []
for _g in range(NGRP):
    cols = range(_g * L, (_g + 1) * L)
    fs = [FIELD_OF_COL[c] for c in cols]
    _FMAP.append(tuple(fs))
    _DVEC.append(tuple(DIMS[f] for f in fs))
    _BJVEC.append(tuple(TBL_BASE[f] + (c - COL_OFF[f])
                        for c, f in zip(cols, fs)))

_CONSTS = tuple(_FMAP) + tuple(_DVEC) + tuple(_BJVEC)
NCONST = len(_CONSTS)

_mesh = plsc.VectorSubcoreMesh(core_axis_name="c", subcore_axis_name="s")


@functools.partial(
    pl.kernel,
    out_type=jax.ShapeDtypeStruct((B, D_OUT), jnp.float32),
    mesh=_mesh,
    compiler_params=pltpu.CompilerParams(
        needs_layout_passes=False, use_tc_tiling_on_sc=False),
    scratch_types=[
        pltpu.VMEM((CHUNK, NF), jnp.int32),
        pltpu.VMEM((TBL_PAD,), jnp.float32),
        pltpu.VMEM((CHUNK, D_OUT), jnp.float32),
        pltpu.VMEM((NCONST, L), jnp.int32),
    ],
)
def _embed_sc(idx_hbm, tbl_hbm, cmat_hbm, out_hbm, stage_v, tbl_v, out_v,
              cmat_v):
    wid = lax.axis_index("s") * NC + lax.axis_index("c")
    base = wid * CHUNK

    with jax.named_scope("stage_in"):
        pltpu.sync_copy(tbl_hbm, tbl_v)
        pltpu.sync_copy(cmat_hbm, cmat_v)
        pltpu.sync_copy(idx_hbm.at[pl.ds(base, CHUNK)], stage_v)

    fmap = [cmat_v[g] for g in range(NGRP)]
    dvec = [cmat_v[NGRP + g] for g in range(NGRP)]
    bjvec = [cmat_v[2 * NGRP + g] for g in range(NGRP)]

    with jax.named_scope("row_loop"):
        _do_rows(stage_v, tbl_v, out_v, fmap, dvec, bjvec)
    with jax.named_scope("write_out"):
        pltpu.sync_copy(out_v, out_hbm.at[pl.ds(base, CHUNK)])


def _do_rows(stage_v, tbl_v, out_v, fmap, dvec, bjvec):
    @plsc.parallel_loop(0, CHUNK, unroll=UNROLL)
    def _row_loop(b):
        bsplat = jnp.full((L,), b, jnp.int32)
        for g in range(NGRP):
            raw = plsc.load_gather(stage_v, [bsplat, fmap[g]])
            tidx = raw * dvec[g] + bjvec[g]
            vals = plsc.load_gather(tbl_v, [tidx])
            out_v[b, pl.ds(g * L, L)] = vals


def kernel(type1, type2, primary_color, secondary_color, shape, size,
           evolution_stage, habitat, legendary, mythical,
           type1_table, type2_table, primary_color_table,
           secondary_color_table, shape_table, size_table,
           evolution_stage_table, habitat_table, legendary_table,
           mythical_table):
    idx2 = jnp.stack(
        [x.astype(jnp.int32) for x in
         (type1, type2, primary_color, secondary_color, shape, size,
          evolution_stage, habitat, legendary, mythical)], axis=1)
    tables = (type1_table, type2_table, primary_color_table,
              secondary_color_table, shape_table, size_table,
              evolution_stage_table, habitat_table, legendary_table,
              mythical_table)
    tbl_flat = jnp.concatenate([t.reshape(-1) for t in tables])
    tbl_flat = jnp.pad(tbl_flat, (0, TBL_PAD - TBL_WORDS))
    cmat = jnp.asarray(_CONSTS, dtype=jnp.int32)
    return _embed_sc(idx2, tbl_flat, cmat)


# speedup vs baseline: 1.4993x; 1.1354x over previous
import functools

import jax
import jax.numpy as jnp
from jax import lax
from jax.experimental import pallas as pl
from jax.experimental.pallas import tpu as pltpu
from jax.experimental.pallas import tpu_sc as plsc

B = 16384
DIMS = (10, 10, 8, 8, 6, 6, 6, 6, 2, 2)
ROWS = (18, 19, 10, 11, 14, 6, 3, 9, 2, 2)
NF = len(DIMS)
NFP = 11
D_OUT = sum(DIMS)

COL_OFF = []
_acc = 0
for _d in DIMS:
    COL_OFF.append(_acc)
    _acc += _d

TBL_BASE = []
_acc = 0
for _n, _d in zip(ROWS, DIMS):
    TBL_BASE.append(_acc)
    _acc += _n * _d
TBL_WORDS = _acc
TBL_PAD = (TBL_WORDS + 7) // 8 * 8

FIELD_OF_COL = []
for _f, _d in enumerate(DIMS):
    FIELD_OF_COL.extend([_f] * _d)

NC, NS, L = 2, 16, 16
NW = NC * NS
CHUNK = B // NW
NGRP = D_OUT // L
UNROLL = 4

_FMAP, _DVEC, _BJVEC = [], [], []
for _g in range(NGRP):
    cols = range(_g * L, (_g + 1) * L)
    fs = [FIELD_OF_COL[c] for c in cols]
    _FMAP.append(tuple(fs))
    _DVEC.append(tuple(DIMS[f] for f in fs))
    _BJVEC.append(tuple(TBL_BASE[f] + (c - COL_OFF[f])
                        for c, f in zip(cols, fs)))

_CONSTS = tuple(_FMAP) + tuple(_DVEC) + tuple(_BJVEC)
NCONST = len(_CONSTS)

_mesh = plsc.VectorSubcoreMesh(core_axis_name="c", subcore_axis_name="s")


@functools.partial(
    pl.kernel,
    out_type=jax.ShapeDtypeStruct((B * D_OUT,), jnp.float32),
    mesh=_mesh,
    compiler_params=pltpu.CompilerParams(
        needs_layout_passes=False, use_tc_tiling_on_sc=False),
    scratch_types=[
        pltpu.VMEM((NF, CHUNK), jnp.int32),
        pltpu.VMEM((CHUNK * NFP,), jnp.int32),
        pltpu.VMEM((TBL_PAD,), jnp.float32),
        pltpu.VMEM((CHUNK * D_OUT,), jnp.float32),
        pltpu.VMEM((NCONST, L), jnp.int32),
    ],
)
def _embed_sc(i0, i1, i2, i3, i4, i5, i6, i7, i8, i9, tbl_hbm, cmat_hbm,
              out_hbm, stage_v, pack_v, tbl_v, out_v, cmat_v):
    wid = lax.axis_index("s") * NC + lax.axis_index("c")
    base = wid * CHUNK

    with jax.named_scope("stage_in"):
        pltpu.sync_copy(tbl_hbm, tbl_v)
        pltpu.sync_copy(cmat_hbm, cmat_v)
        idx_refs = (i0, i1, i2, i3, i4, i5, i6, i7, i8, i9)
        for f in range(NF):
            pltpu.sync_copy(idx_refs[f].at[pl.ds(base, CHUNK)],
                            stage_v.at[f])

    fmap = [cmat_v[g] for g in range(NGRP)]
    dvec = [cmat_v[NGRP + g] for g in range(NGRP)]
    bjvec = [cmat_v[2 * NGRP + g] for g in range(NGRP)]

    with jax.named_scope("repack"):
        iota_nfp = lax.iota(jnp.int32, L) * NFP
        for f in range(NF):
            @plsc.parallel_loop(0, CHUNK // L, unroll=4)
            def _repack(g, f=f):
                v = stage_v[f, pl.ds(g * L, L)]
                dst = iota_nfp + (g * (L * NFP) + f)
                plsc.store_scatter(pack_v, [dst], v)

    with jax.named_scope("row_loop"):
        @plsc.parallel_loop(0, CHUNK, unroll=UNROLL)
        def _row_loop(b):
            bsplat = jnp.full((L,), b * NFP, jnp.int32)
            for g in range(NGRP):
                raw = plsc.load_gather(pack_v, [bsplat + fmap[g]])
                tidx = raw * dvec[g] + bjvec[g]
                vals = plsc.load_gather(tbl_v, [tidx])
                out_v[pl.ds(b * D_OUT + g * L, L)] = vals

    with jax.named_scope("write_out"):
        pltpu.sync_copy(out_v, out_hbm.at[pl.ds(base * D_OUT,
                                                CHUNK * D_OUT)])


def kernel(type1, type2, primary_color, secondary_color, shape, size,
           evolution_stage, habitat, legendary, mythical,
           type1_table, type2_table, primary_color_table,
           secondary_color_table, shape_table, size_table,
           evolution_stage_table, habitat_table, legendary_table,
           mythical_table):
    idxs = [x.astype(jnp.int32) for x in
            (type1, type2, primary_color, secondary_color, shape, size,
             evolution_stage, habitat, legendary, mythical)]
    tables = (type1_table, type2_table, primary_color_table,
              secondary_color_table, shape_table, size_table,
              evolution_stage_table, habitat_table, legendary_table,
              mythical_table)
    tbl_flat = jnp.concatenate([t.reshape(-1) for t in tables])
    tbl_flat = jnp.pad(tbl_flat, (0, TBL_PAD - TBL_WORDS))
    cmat = jnp.asarray(_CONSTS, dtype=jnp.int32)
    out = _embed_sc(*idxs, tbl_flat, cmat)
    return out.reshape(B, D_OUT)


# speedup vs baseline: 1.5945x; 1.0635x over previous
import functools

import jax
import jax.numpy as jnp
from jax import lax
from jax.experimental import pallas as pl
from jax.experimental.pallas import tpu as pltpu
from jax.experimental.pallas import tpu_sc as plsc

B = 16384
DIMS = (10, 10, 8, 8, 6, 6, 6, 6, 2, 2)
ROWS = (18, 19, 10, 11, 14, 6, 3, 9, 2, 2)
NF = len(DIMS)
NFP = 11
D_OUT = sum(DIMS)

COL_OFF = []
_acc = 0
for _d in DIMS:
    COL_OFF.append(_acc)
    _acc += _d

TBL_BASE = []
_acc = 0
for _n, _d in zip(ROWS, DIMS):
    TBL_BASE.append(_acc)
    _acc += _n * _d
TBL_WORDS = _acc
TBL_PAD = (TBL_WORDS + 7) // 8 * 8

FIELD_OF_COL = []
for _f, _d in enumerate(DIMS):
    FIELD_OF_COL.extend([_f] * _d)

NC, NS, L = 2, 16, 16
NW = NC * NS
CHUNK = B // NW
NGRP = D_OUT // L
UNROLL = 4

_FMAP, _DVEC, _BJVEC = [], [], []
for _g in range(NGRP):
    cols = range(_g * L, (_g + 1) * L)
    fs = [FIELD_OF_COL[c] for c in cols]
    _FMAP.append(tuple(fs))
    _DVEC.append(tuple(DIMS[f] for f in fs))
    _BJVEC.append(tuple(TBL_BASE[f] + (c - COL_OFF[f])
                        for c, f in zip(cols, fs)))

_CONSTS = tuple(_FMAP) + tuple(_DVEC) + tuple(_BJVEC)
NCONST = len(_CONSTS)
CM_WORDS = NCONST * L
ICAT_LEN = NF * B + CM_WORDS

_mesh = plsc.VectorSubcoreMesh(core_axis_name="c", subcore_axis_name="s")


@functools.partial(
    pl.kernel,
    out_type=jax.ShapeDtypeStruct((B, D_OUT), jnp.float32),
    mesh=_mesh,
    compiler_params=pltpu.CompilerParams(
        needs_layout_passes=False, use_tc_tiling_on_sc=True),
    scratch_types=[
        pltpu.VMEM((NF, CHUNK), jnp.int32),
        pltpu.VMEM((CHUNK * NFP,), jnp.int32),
        pltpu.VMEM((TBL_PAD,), jnp.float32),
        pltpu.VMEM((CHUNK, D_OUT), jnp.float32),
        pltpu.VMEM((CM_WORDS,), jnp.int32),
        pltpu.SemaphoreType.DMA,
    ],
)
def _embed_sc(icat_hbm, tbl_hbm, out_hbm, stage_v, pack_v, tbl_v, out_v,
              cmat_v, sem):
    wid = lax.axis_index("s") * NC + lax.axis_index("c")
    base = wid * CHUNK

    with jax.named_scope("stage_in"):
        copies = [
            pltpu.make_async_copy(tbl_hbm, tbl_v, sem),
            pltpu.make_async_copy(
                icat_hbm.at[pl.ds(NF * B, CM_WORDS)], cmat_v, sem),
        ]
        copies += [
            pltpu.make_async_copy(
                icat_hbm.at[pl.ds(f * B + base, CHUNK)], stage_v.at[f],
                sem)
            for f in range(NF)
        ]
        for c in copies:
            c.start()
        for c in copies:
            c.wait()

    fmap = [cmat_v[pl.ds(g * L, L)] for g in range(NGRP)]
    dvec = [cmat_v[pl.ds((NGRP + g) * L, L)] for g in range(NGRP)]
    bjvec = [cmat_v[pl.ds((2 * NGRP + g) * L, L)] for g in range(NGRP)]

    with jax.named_scope("repack"):
        iota_nfp = lax.iota(jnp.int32, L) * NFP
        for f in range(NF):
            @plsc.parallel_loop(0, CHUNK // L, unroll=4)
            def _repack(g, f=f):
                v = stage_v[f, pl.ds(g * L, L)]
                dst = iota_nfp + (g * (L * NFP) + f)
                plsc.store_scatter(pack_v, [dst], v)

    with jax.named_scope("row_loop"):
        @plsc.parallel_loop(0, CHUNK, unroll=UNROLL)
        def _row_loop(b):
            bsplat = jnp.full((L,), b * NFP, jnp.int32)
            for g in range(NGRP):
                raw = plsc.load_gather(pack_v, [bsplat + fmap[g]])
                tidx = raw * dvec[g] + bjvec[g]
                vals = plsc.load_gather(tbl_v, [tidx])
                out_v[b, pl.ds(g * L, L)] = vals

    with jax.named_scope("write_out"):
        pltpu.sync_copy(out_v, out_hbm.at[pl.ds(base, CHUNK)])


def kernel(type1, type2, primary_color, secondary_color, shape, size,
           evolution_stage, habitat, legendary, mythical,
           type1_table, type2_table, primary_color_table,
           secondary_color_table, shape_table, size_table,
           evolution_stage_table, habitat_table, legendary_table,
           mythical_table):
    idxs = [x.astype(jnp.int32) for x in
            (type1, type2, primary_color, secondary_color, shape, size,
             evolution_stage, habitat, legendary, mythical)]
    cmat = jnp.asarray(_CONSTS, dtype=jnp.int32)
    icat = jnp.concatenate(idxs + [cmat.reshape(-1)])
    tables = (type1_table, type2_table, primary_color_table,
              secondary_color_table, shape_table, size_table,
              evolution_stage_table, habitat_table, legendary_table,
              mythical_table)
    tbl_flat = jnp.concatenate([t.reshape(-1) for t in tables])
    tbl_flat = jnp.pad(tbl_flat, (0, TBL_PAD - TBL_WORDS))
    return _embed_sc(icat, tbl_flat)


# speedup vs baseline: 2.4218x; 1.5189x over previous
import functools

import jax
import jax.numpy as jnp
from jax import lax
from jax.experimental import pallas as pl
from jax.experimental.pallas import tpu as pltpu
from jax.experimental.pallas import tpu_sc as plsc

B = 16384
DIMS = (10, 10, 8, 8, 6, 6, 6, 6, 2, 2)
ROWS = (18, 19, 10, 11, 14, 6, 3, 9, 2, 2)
NF = len(DIMS)
D_OUT = sum(DIMS)

COL_OFF = []
_acc = 0
for _d in DIMS:
    COL_OFF.append(_acc)
    _acc += _d

FIELD_OF_COL = []
for _f, _d in enumerate(DIMS):
    FIELD_OF_COL.extend([_f] * _d)

NC, NS, L = 2, 16, 16
NW = NC * NS
CHUNK = B // NW
UNROLL = 2

_mesh = plsc.VectorSubcoreMesh(core_axis_name="c", subcore_axis_name="s")


@functools.partial(
    pl.kernel,
    out_type=jax.ShapeDtypeStruct((D_OUT, B), jnp.float32),
    mesh=_mesh,
    compiler_params=pltpu.CompilerParams(
        needs_layout_passes=False, use_tc_tiling_on_sc=True),
    scratch_types=[
        pltpu.VMEM((NF, CHUNK), jnp.int32),
        pltpu.VMEM((D_OUT, CHUNK), jnp.float32),
        [pltpu.VMEM((d, n), jnp.float32) for n, d in zip(ROWS, DIMS)],
        pltpu.SemaphoreType.DMA,
    ],
)
def _embed_sc(i0, i1, i2, i3, i4, i5, i6, i7, i8, i9,
              t0, t1, t2, t3, t4, t5, t6, t7, t8, t9,
              out_hbm, stage_v, out_v, tbl_vs, sem):
    wid = lax.axis_index("s") * NC + lax.axis_index("c")
    base = wid * CHUNK

    with jax.named_scope("stage_in"):
        idx_refs = (i0, i1, i2, i3, i4, i5, i6, i7, i8, i9)
        tbl_refs = (t0, t1, t2, t3, t4, t5, t6, t7, t8, t9)
        copies = [pltpu.make_async_copy(tbl_refs[f], tbl_vs[f], sem)
                  for f in range(NF)]
        copies += [
            pltpu.make_async_copy(
                idx_refs[f].at[pl.ds(base, CHUNK)], stage_v.at[f], sem)
            for f in range(NF)
        ]
        for c in copies:
            c.start()
        for c in copies:
            c.wait()

    with jax.named_scope("col_loop"):
        @plsc.parallel_loop(0, CHUNK // L, unroll=UNROLL)
        def _grp_loop(g):
            r0 = g * L
            raws = [stage_v[f, pl.ds(r0, L)] for f in range(NF)]
            for c in range(D_OUT):
                f = FIELD_OF_COL[c]
                j = c - COL_OFF[f]
                jsplat = jnp.broadcast_to(jnp.int32(j), (L,))
                vals = plsc.load_gather(tbl_vs[f], [jsplat, raws[f]])
                out_v[c, pl.ds(r0, L)] = vals

    with jax.named_scope("write_out"):
        pltpu.sync_copy(out_v, out_hbm.at[:, pl.ds(base, CHUNK)])


def kernel(type1, type2, primary_color, secondary_color, shape, size,
           evolution_stage, habitat, legendary, mythical,
           type1_table, type2_table, primary_color_table,
           secondary_color_table, shape_table, size_table,
           evolution_stage_table, habitat_table, legendary_table,
           mythical_table):
    idxs = [x.astype(jnp.int32) for x in
            (type1, type2, primary_color, secondary_color, shape, size,
             evolution_stage, habitat, legendary, mythical)]
    tables = (type1_table, type2_table, primary_color_table,
              secondary_color_table, shape_table, size_table,
              evolution_stage_table, habitat_table, legendary_table,
              mythical_table)
    out_t = _embed_sc(*idxs, *[t.T for t in tables])
    return out_t.T
